# baseline (speedup 1.0000x reference)
import functools

import jax
import jax.numpy as jnp
from jax import lax
from jax.experimental import pallas as pl
from jax.experimental.pallas import tpu as pltpu
from jax.experimental.pallas import tpu_sc as plsc

N_ATOMS = 10240
N_BONDS = 327680
MAX_NB = 32
ATOM_FDIM = 128
BOND_FDIM = 16
H = 128
DEPTH = 4
B_MOLS = 64

NC, NS = 2, 16
NW = NC * NS
HV = H // 16

_mesh = plsc.VectorSubcoreMesh(core_axis_name="c", subcore_axis_name="s")

_CA = 1
_APW = N_ATOMS // NW
_AGG_NCH = _APW // _CA
_AGG_ROWS = _CA * MAX_NB
_ANB = 4


@functools.partial(
    pl.kernel,
    out_type=jax.ShapeDtypeStruct((N_ATOMS, H), jnp.float32),
    mesh=_mesh,
    scratch_types=(
        [pltpu.VMEM((_APW * MAX_NB,), jnp.int32)]
        + [pltpu.VMEM((_AGG_ROWS, H), jnp.float32)] * _ANB
        + [pltpu.VMEM((_APW, H), jnp.float32)]
        + [pltpu.SemaphoreType.DMA] * _ANB
    ),
)
def _sc_agg(a2b_hbm, msg_hbm, agg_hbm, idx_all, *rest):
    wid = lax.axis_index("s") * NC + lax.axis_index("c")
    rows = rest[0:_ANB]
    out_v = rest[_ANB]
    sems = rest[_ANB + 1:]

    def _fire(ci, s):
        pltpu.async_copy(
            msg_hbm.at[idx_all.at[pl.ds(ci * _AGG_ROWS, _AGG_ROWS)]],
            rows[s], sems[s])

    def _wait(s):
        pltpu.make_async_copy(msg_hbm.at[pl.ds(0, _AGG_ROWS)], rows[s],
                              sems[s]).wait()

    def _compute(ci, s):
        for a in range(_CA):
            base = a * MAX_NB
            for h in range(HV):
                sl = pl.ds(16 * h, 16)
                v = rows[s][base, sl]
                acc_s = v
                acc_m = v
                for nb in range(1, MAX_NB):
                    v = rows[s][base + nb, sl]
                    acc_s = acc_s + v
                    acc_m = jnp.maximum(acc_m, v)
                out_v[ci * _CA + a, sl] = acc_s * acc_m

    pltpu.sync_copy(a2b_hbm.at[pl.ds(wid * _APW * MAX_NB, _APW * MAX_NB)],
                    idx_all)
    for p in range(_ANB - 1):
        _fire(p, p)

    @pl.loop(0, _AGG_NCH, step=_ANB)
    def _chunk(ci):
        for s in range(_ANB):
            cur = ci + s

            @pl.when(cur + _ANB - 1 < _AGG_NCH)
            def _():
                _fire(cur + _ANB - 1, (s + _ANB - 1) % _ANB)

            _wait(s)
            _compute(cur, s)

    pltpu.sync_copy(out_v, agg_hbm.at[pl.ds(wid * _APW, _APW)])


_CB = 128
_BPW = N_BONDS // NW
_BOND_NCH = _BPW // _CB
_BNB = 2


@functools.partial(
    pl.kernel,
    out_type=jax.ShapeDtypeStruct((N_BONDS, H), jnp.float32),
    mesh=_mesh,
    scratch_types=(
        [pltpu.VMEM((_BPW,), jnp.int32)] * 2
        + [pltpu.VMEM((_CB, H), jnp.float32)] * (3 * _BNB)
        + [pltpu.SemaphoreType.DMA] * (4 * _BNB)
    ),
)
def _sc_bond(b2a_hbm, b2revb_hbm, a_hbm, r_hbm, ib_hbm, out_hbm,
             ba_all, br_all, *bufs_and_sems):
    wid = lax.axis_index("s") * NC + lax.axis_index("c")
    b_base = wid * _BPW
    arows = bufs_and_sems[0:_BNB]
    rrows = bufs_and_sems[_BNB:2 * _BNB]
    ibrows = bufs_and_sems[2 * _BNB:3 * _BNB]
    sems = bufs_and_sems[3 * _BNB:]
    sem_a = sems[0:_BNB]
    sem_r = sems[_BNB:2 * _BNB]
    sem_i = sems[2 * _BNB:3 * _BNB]
    sem_o = sems[3 * _BNB:4 * _BNB]

    def _fire_gathers(ci, s):
        pltpu.async_copy(a_hbm.at[ba_all.at[pl.ds(ci * _CB, _CB)]],
                         arows[s], sem_a[s])
        pltpu.async_copy(r_hbm.at[br_all.at[pl.ds(ci * _CB, _CB)]],
                         rrows[s], sem_r[s])

    def _fire_ib(ci, s):
        pltpu.async_copy(ib_hbm.at[pl.ds(b_base + ci * _CB, _CB)],
                         ibrows[s], sem_i[s])

    def _fire_store(ci, s):
        pltpu.async_copy(ibrows[s],
                         out_hbm.at[pl.ds(b_base + ci * _CB, _CB)],
                         sem_o[s])

    def _wait(dst, sem):
        pltpu.make_async_copy(ib_hbm.at[pl.ds(0, _CB)], dst, sem).wait()

    def _wait_store(s):
        pltpu.make_async_copy(ibrows[s], out_hbm.at[pl.ds(0, _CB)],
                              sem_o[s]).wait()

    pltpu.sync_copy(b2a_hbm.at[pl.ds(b_base, _BPW)], ba_all)
    pltpu.sync_copy(b2revb_hbm.at[pl.ds(b_base, _BPW)], br_all)
    for p in range(_BNB - 1):
        _fire_gathers(p, p)
        _fire_ib(p, p)

    @pl.loop(0, _BOND_NCH, step=_BNB)
    def _chunk(ci):
        for s in range(_BNB):
            cur = ci + s
            nxt_s = (s + _BNB - 1) % _BNB

            @pl.when(cur + _BNB - 1 < _BOND_NCH)
            def _():
                _fire_gathers(cur + _BNB - 1, nxt_s)

                @pl.when(cur >= 1)
                def _():
                    _wait_store(nxt_s)

                _fire_ib(cur + _BNB - 1, nxt_s)

            _wait(arows[s], sem_a[s])
            _wait(rrows[s], sem_r[s])
            _wait(ibrows[s], sem_i[s])

            @pl.loop(0, _CB, unroll=4)
            def _row(i):
                for h in range(HV):
                    sl = pl.ds(16 * h, 16)
                    v = ibrows[s][i, sl] + arows[s][i, sl] - rrows[s][i, sl]
                    ibrows[s][i, sl] = jnp.maximum(v, 0.0)

            _fire_store(cur, s)

    for s in range(_BNB):
        _wait_store(s)


def _relu_matmul_body(x_ref, w_ref, o_ref):
    o_ref[...] = jnp.maximum(
        jnp.dot(x_ref[...], w_ref[...], preferred_element_type=jnp.float32), 0.0)


def _matmul_body(x_ref, w_ref, o_ref):
    o_ref[...] = jnp.dot(x_ref[...], w_ref[...], preferred_element_type=jnp.float32)


def _row_blocked(body, n_rows, k, block):
    return pl.pallas_call(
        body,
        grid=(n_rows // block,),
        in_specs=[
            pl.BlockSpec((block, k), lambda i: (i, 0)),
            pl.BlockSpec((k, H), lambda i: (0, 0)),
        ],
        out_specs=pl.BlockSpec((block, H), lambda i: (i, 0)),
        out_shape=jax.ShapeDtypeStruct((n_rows, H), jnp.float32),
    )


_tc_proj_atom = _row_blocked(_relu_matmul_body, N_ATOMS, ATOM_FDIM, 2048)
_tc_proj_bond = _row_blocked(_relu_matmul_body, N_BONDS, BOND_FDIM, 4096)
_tc_matmul_big = _row_blocked(_matmul_body, N_BONDS, H, 2048)


def _atom_update_body(ma_ref, agg_ref, w_ref, man_ref, a_ref):
    man = ma_ref[...] + agg_ref[...]
    man_ref[...] = man
    a_ref[...] = jnp.dot(man, w_ref[...], preferred_element_type=jnp.float32)


_tc_atom_update = pl.pallas_call(
    _atom_update_body,
    out_shape=(jax.ShapeDtypeStruct((N_ATOMS, H), jnp.float32),
               jax.ShapeDtypeStruct((N_ATOMS, H), jnp.float32)),
)


def _readout_body(agg_ref, ma_ref, ia_ref, w1_ref, w2_ref, w3_ref,
                  wo_ref, b_ref, inv_ref, o_ref):
    h = jnp.dot(agg_ref[...], w1_ref[...], preferred_element_type=jnp.float32)
    h = h + jnp.dot(ma_ref[...], w2_ref[...], preferred_element_type=jnp.float32)
    h = h + jnp.dot(ia_ref[...], w3_ref[...], preferred_element_type=jnp.float32)
    ah = jnp.maximum(
        jnp.dot(h, wo_ref[...], preferred_element_type=jnp.float32) + b_ref[...],
        0.0)
    per = N_ATOMS // B_MOLS
    sums = ah.reshape(B_MOLS, per, H).sum(axis=1)
    o_ref[...] = sums * inv_ref[...]


_tc_readout = pl.pallas_call(
    _readout_body,
    out_shape=jax.ShapeDtypeStruct((B_MOLS, H), jnp.float32),
)


def kernel(f_atoms, f_bonds, a2b, b2a, b2revb, a_scope,
           W_i_atom, W_i_bond, W_h, lr_W, W_o_W, W_o_b):
    ia = _tc_proj_atom(f_atoms, W_i_atom)
    ib = _tc_proj_bond(f_bonds, W_i_bond)
    a2b_flat = a2b.reshape(-1)

    ma, msg = ia, ib
    for d in range(DEPTH - 1):
        agg = _sc_agg(a2b_flat, msg)
        Rm = _tc_matmul_big(msg, W_h[d])
        ma, A = _tc_atom_update(ma, agg, W_h[d])
        msg = _sc_bond(b2a, b2revb, A, Rm, ib)

    agg3 = _sc_agg(a2b_flat, msg)
    inv = (1.0 / a_scope[:, 1].astype(jnp.float32)).reshape(B_MOLS, 1)
    return _tc_readout(agg3, ma, ia,
                       lr_W[:H], lr_W[H:2 * H], lr_W[2 * H:],
                       W_o_W, W_o_b.reshape(1, H), inv)


# speedup vs baseline: 1.0611x; 1.0611x over previous
import functools

import jax
import jax.numpy as jnp
from jax import lax
from jax.experimental import pallas as pl
from jax.experimental.pallas import tpu as pltpu
from jax.experimental.pallas import tpu_sc as plsc

N_ATOMS = 10240
N_BONDS = 327680
MAX_NB = 32
ATOM_FDIM = 128
BOND_FDIM = 16
H = 128
DEPTH = 4
B_MOLS = 64

NC, NS = 2, 16
NW = NC * NS
HV = H // 16

_mesh = plsc.VectorSubcoreMesh(core_axis_name="c", subcore_axis_name="s")

_CA = 1
_APW = N_ATOMS // NW
_AGG_NCH = _APW // _CA
_AGG_ROWS = _CA * MAX_NB
_ANB = 2


@functools.partial(
    pl.kernel,
    out_type=jax.ShapeDtypeStruct((N_ATOMS, H), jnp.float32),
    mesh=_mesh,
    scratch_types=(
        [pltpu.VMEM((_APW * MAX_NB,), jnp.int32)]
        + [pltpu.VMEM((_AGG_ROWS, H), jnp.float32)] * _ANB
        + [pltpu.VMEM((_APW, H), jnp.float32)]
        + [pltpu.SemaphoreType.DMA] * _ANB
    ),
)
def _sc_agg(a2b_hbm, msg_hbm, agg_hbm, idx_all, *rest):
    wid = lax.axis_index("s") * NC + lax.axis_index("c")
    rows = rest[0:_ANB]
    out_v = rest[_ANB]
    sems = rest[_ANB + 1:]

    def _fire(ci, s):
        pltpu.async_copy(
            msg_hbm.at[idx_all.at[pl.ds(ci * _AGG_ROWS, _AGG_ROWS)]],
            rows[s], sems[s])

    def _wait(s):
        pltpu.make_async_copy(msg_hbm.at[pl.ds(0, _AGG_ROWS)], rows[s],
                              sems[s]).wait()

    def _compute(ci, s):
        for a in range(_CA):
            base = a * MAX_NB
            for h in range(HV):
                sl = pl.ds(16 * h, 16)
                v = rows[s][base, sl]
                acc_s = v
                acc_m = v
                for nb in range(1, MAX_NB):
                    v = rows[s][base + nb, sl]
                    acc_s = acc_s + v
                    acc_m = jnp.maximum(acc_m, v)
                out_v[ci * _CA + a, sl] = acc_s * acc_m

    pltpu.sync_copy(a2b_hbm.at[pl.ds(wid * _APW * MAX_NB, _APW * MAX_NB)],
                    idx_all)
    for p in range(_ANB - 1):
        _fire(p, p)

    @pl.loop(0, _AGG_NCH, step=_ANB)
    def _chunk(ci):
        for s in range(_ANB):
            cur = ci + s

            @pl.when(cur + _ANB - 1 < _AGG_NCH)
            def _():
                _fire(cur + _ANB - 1, (s + _ANB - 1) % _ANB)

            _wait(s)
            _compute(cur, s)

    pltpu.sync_copy(out_v, agg_hbm.at[pl.ds(wid * _APW, _APW)])


_CB = 128
_BPW = N_BONDS // NW
_BOND_NCH = _BPW // _CB
_BNB = 2


@functools.partial(
    pl.kernel,
    out_type=jax.ShapeDtypeStruct((N_BONDS, H), jnp.float32),
    mesh=_mesh,
    scratch_types=(
        [pltpu.VMEM((_BPW,), jnp.int32)] * 2
        + [pltpu.VMEM((_CB, H), jnp.float32)] * (3 * _BNB)
        + [pltpu.SemaphoreType.DMA] * (4 * _BNB)
    ),
)
def _sc_bond(b2a_hbm, b2revb_hbm, a_hbm, r_hbm, ib_hbm, out_hbm,
             ba_all, br_all, *bufs_and_sems):
    wid = lax.axis_index("s") * NC + lax.axis_index("c")
    b_base = wid * _BPW
    arows = bufs_and_sems[0:_BNB]
    rrows = bufs_and_sems[_BNB:2 * _BNB]
    ibrows = bufs_and_sems[2 * _BNB:3 * _BNB]
    sems = bufs_and_sems[3 * _BNB:]
    sem_a = sems[0:_BNB]
    sem_r = sems[_BNB:2 * _BNB]
    sem_i = sems[2 * _BNB:3 * _BNB]
    sem_o = sems[3 * _BNB:4 * _BNB]

    def _fire_gathers(ci, s):
        for k in range(_CB // 32):
            sub = pl.ds(ci * _CB + 32 * k, 32)
            dst = pl.ds(32 * k, 32)
            pltpu.async_copy(a_hbm.at[ba_all.at[sub]],
                             arows[s].at[dst], sem_a[s])
            pltpu.async_copy(r_hbm.at[br_all.at[sub]],
                             rrows[s].at[dst], sem_r[s])

    def _fire_ib(ci, s):
        pltpu.async_copy(ib_hbm.at[pl.ds(b_base + ci * _CB, _CB)],
                         ibrows[s], sem_i[s])

    def _fire_store(ci, s):
        pltpu.async_copy(ibrows[s],
                         out_hbm.at[pl.ds(b_base + ci * _CB, _CB)],
                         sem_o[s])

    def _wait(dst, sem):
        pltpu.make_async_copy(ib_hbm.at[pl.ds(0, _CB)], dst, sem).wait()

    def _wait_store(s):
        pltpu.make_async_copy(ibrows[s], out_hbm.at[pl.ds(0, _CB)],
                              sem_o[s]).wait()

    pltpu.sync_copy(b2a_hbm.at[pl.ds(b_base, _BPW)], ba_all)
    pltpu.sync_copy(b2revb_hbm.at[pl.ds(b_base, _BPW)], br_all)
    for p in range(_BNB - 1):
        _fire_gathers(p, p)
        _fire_ib(p, p)

    @pl.loop(0, _BOND_NCH, step=_BNB)
    def _chunk(ci):
        for s in range(_BNB):
            cur = ci + s
            nxt_s = (s + _BNB - 1) % _BNB

            @pl.when(cur + _BNB - 1 < _BOND_NCH)
            def _():
                _fire_gathers(cur + _BNB - 1, nxt_s)

                @pl.when(cur >= 1)
                def _():
                    _wait_store(nxt_s)

                _fire_ib(cur + _BNB - 1, nxt_s)

            _wait(arows[s], sem_a[s])
            _wait(rrows[s], sem_r[s])
            _wait(ibrows[s], sem_i[s])

            @pl.loop(0, _CB, unroll=4)
            def _row(i):
                for h in range(HV):
                    sl = pl.ds(16 * h, 16)
                    v = ibrows[s][i, sl] + arows[s][i, sl] - rrows[s][i, sl]
                    ibrows[s][i, sl] = jnp.maximum(v, 0.0)

            _fire_store(cur, s)

    for s in range(_BNB):
        _wait_store(s)


def _relu_matmul_body(x_ref, w_ref, o_ref):
    o_ref[...] = jnp.maximum(
        jnp.dot(x_ref[...], w_ref[...], preferred_element_type=jnp.float32), 0.0)


def _matmul_body(x_ref, w_ref, o_ref):
    o_ref[...] = jnp.dot(x_ref[...], w_ref[...], preferred_element_type=jnp.float32)


def _row_blocked(body, n_rows, k, block):
    return pl.pallas_call(
        body,
        grid=(n_rows // block,),
        in_specs=[
            pl.BlockSpec((block, k), lambda i: (i, 0)),
            pl.BlockSpec((k, H), lambda i: (0, 0)),
        ],
        out_specs=pl.BlockSpec((block, H), lambda i: (i, 0)),
        out_shape=jax.ShapeDtypeStruct((n_rows, H), jnp.float32),
    )


_tc_proj_atom = _row_blocked(_relu_matmul_body, N_ATOMS, ATOM_FDIM, 2048)
_tc_proj_bond = _row_blocked(_relu_matmul_body, N_BONDS, BOND_FDIM, 4096)
_tc_matmul_big = _row_blocked(_matmul_body, N_BONDS, H, 2048)


def _atom_update_body(ma_ref, agg_ref, w_ref, man_ref, a_ref):
    man = ma_ref[...] + agg_ref[...]
    man_ref[...] = man
    a_ref[...] = jnp.dot(man, w_ref[...], preferred_element_type=jnp.float32)


_tc_atom_update = pl.pallas_call(
    _atom_update_body,
    out_shape=(jax.ShapeDtypeStruct((N_ATOMS, H), jnp.float32),
               jax.ShapeDtypeStruct((N_ATOMS, H), jnp.float32)),
)


def _readout_body(agg_ref, ma_ref, ia_ref, w1_ref, w2_ref, w3_ref,
                  wo_ref, b_ref, inv_ref, o_ref):
    h = jnp.dot(agg_ref[...], w1_ref[...], preferred_element_type=jnp.float32)
    h = h + jnp.dot(ma_ref[...], w2_ref[...], preferred_element_type=jnp.float32)
    h = h + jnp.dot(ia_ref[...], w3_ref[...], preferred_element_type=jnp.float32)
    ah = jnp.maximum(
        jnp.dot(h, wo_ref[...], preferred_element_type=jnp.float32) + b_ref[...],
        0.0)
    per = N_ATOMS // B_MOLS
    sums = ah.reshape(B_MOLS, per, H).sum(axis=1)
    o_ref[...] = sums * inv_ref[...]


_tc_readout = pl.pallas_call(
    _readout_body,
    out_shape=jax.ShapeDtypeStruct((B_MOLS, H), jnp.float32),
)


def kernel(f_atoms, f_bonds, a2b, b2a, b2revb, a_scope,
           W_i_atom, W_i_bond, W_h, lr_W, W_o_W, W_o_b):
    ia = _tc_proj_atom(f_atoms, W_i_atom)
    ib = _tc_proj_bond(f_bonds, W_i_bond)
    a2b_flat = a2b.reshape(-1)

    ma, msg = ia, ib
    for d in range(DEPTH - 1):
        agg = _sc_agg(a2b_flat, msg)
        Rm = _tc_matmul_big(msg, W_h[d])
        ma, A = _tc_atom_update(ma, agg, W_h[d])
        msg = _sc_bond(b2a, b2revb, A, Rm, ib)

    agg3 = _sc_agg(a2b_flat, msg)
    inv = (1.0 / a_scope[:, 1].astype(jnp.float32)).reshape(B_MOLS, 1)
    return _tc_readout(agg3, ma, ia,
                       lr_W[:H], lr_W[H:2 * H], lr_W[2 * H:],
                       W_o_W, W_o_b.reshape(1, H), inv)
